# baseline (speedup 1.0000x reference)
import functools

import jax
import jax.numpy as jnp
from jax import lax
from jax.experimental import pallas as pl
from jax.experimental.pallas import tpu as pltpu
from jax.experimental.pallas import tpu_sc as plsc

N = 100000
E = 3200000
D = 16

NC = 2
NS = 16
NW = NC * NS

NP = 102400
NODES_PER_TILE = NP // NS
OUT_CHUNK = 640

RPT = 792
ROWS = NW * RPT
EP = ROWS * 128
GRP = 6
NB = RPT // GRP

IDX_CHUNK = 2112
DEG_CHUNKS = (RPT * 128) // IDX_CHUNK

BR = 2048
GRID = NP // BR

_mesh = plsc.VectorSubcoreMesh(core_axis_name="c", subcore_axis_name="s",
                               num_cores=NC, num_subcores=NS)
_sc_params = pltpu.CompilerParams(needs_layout_passes=False,
                                  use_tc_tiling_on_sc=False)


@functools.partial(
    pl.kernel, mesh=_mesh, compiler_params=_sc_params, name="sc_degree",
    out_type=jax.ShapeDtypeStruct((NW, NP), jnp.float32),
    scratch_types=[
        pltpu.VMEM((NP,), jnp.float32),
        pltpu.VMEM((2, IDX_CHUNK), jnp.int32),
        pltpu.SemaphoreType.DMA,
    ],
)
def _deg_kernel(dst_hbm, zeros_hbm, deg_out, hist, ibuf, isem):
    c = lax.axis_index("c")
    s = lax.axis_index("s")
    wid = c * NS + s

    pltpu.sync_copy(zeros_hbm, hist)
    base = wid * (RPT * 128)
    ones = jnp.full((16,), 1.0, jnp.float32)

    def count(buf):
        def vec_body(i, _):
            for u in range(4):
                idxv = buf[pl.ds((i * 4 + u) * 16, 16)]
                plsc.addupdate_scatter(hist, [idxv], ones)
            return 0

        lax.fori_loop(0, IDX_CHUNK // 64, vec_body, 0)

    def chunk(k):
        return dst_hbm.at[pl.ds(base + k * IDX_CHUNK, IDX_CHUNK)]

    pltpu.sync_copy(chunk(0), ibuf.at[0])
    def pair_body(t, _):
        cpb = pltpu.async_copy(chunk(2 * t + 1), ibuf.at[1], isem)
        count(ibuf.at[0])
        cpb.wait()
        nxt = jnp.minimum(2 * t + 2, DEG_CHUNKS - 1)
        cpa = pltpu.async_copy(chunk(nxt), ibuf.at[0], isem)
        count(ibuf.at[1])
        cpa.wait()
        return 0

    lax.fori_loop(0, DEG_CHUNKS // 2, pair_body, 0)
    pltpu.sync_copy(hist, deg_out.at[wid])


@functools.partial(
    pl.kernel, mesh=_mesh, compiler_params=_sc_params, name="sc_gcn_agg",
    out_type=jax.ShapeDtypeStruct((NC, NP, D), jnp.float32),
    scratch_types=[
        pltpu.VMEM((2, GRP, 128), jnp.int32),
        pltpu.VMEM((2, GRP, 128), jnp.int32),
        pltpu.VMEM((2, GRP * 128, D), jnp.float32),
        pltpu.VMEM_SHARED((NP, D), jnp.float32),
        pltpu.SemaphoreType.DMA,
        pltpu.SemaphoreType.DMA,
        pltpu.SemaphoreType.DMA,
        pltpu.SemaphoreType.DMA,
    ],
)
def _agg_kernel(p_hbm, src_hbm, dst_hbm, zrow_hbm, agg_out,
                sbuf, dbuf, gbuf, acc_sh, gsem0, gsem1, ssem0, ssem1):
    c = lax.axis_index("c")
    s = lax.axis_index("s")
    wid = c * NS + s
    nodeoff = s * NODES_PER_TILE
    stage = gbuf.at[0, pl.ds(0, OUT_CHUNK)]

    pltpu.sync_copy(zrow_hbm, stage)
    for i in range(NODES_PER_TILE // OUT_CHUNK):
        pltpu.sync_copy(stage, acc_sh.at[pl.ds(nodeoff + i * OUT_CHUNK, OUT_CHUNK)])
    plsc.subcore_barrier()

    base_row = wid * RPT

    def fire_gathers(b, row, gsem):
        pltpu.sync_copy(src_hbm.at[pl.ds(row, GRP)], sbuf.at[b])
        pltpu.sync_copy(dst_hbm.at[pl.ds(row, GRP)], dbuf.at[b])
        return [pltpu.async_copy(p_hbm.at[sbuf.at[b, j]],
                                 gbuf.at[b, pl.ds(j * 128, 128)], gsem)
                for j in range(GRP)]

    def fire_scatters(b, ssem):
        return [pltpu.async_copy(gbuf.at[b, pl.ds(j * 128, 128)],
                                 acc_sh.at[dbuf.at[b, j]], ssem, add=True)
                for j in range(GRP)]

    def pair(t, _):
        r0 = base_row + (2 * t) * GRP
        g0 = fire_gathers(0, r0, gsem0)
        g1 = fire_gathers(1, r0 + GRP, gsem1)
        for cp in g0:
            cp.wait()
        s0 = fire_scatters(0, ssem0)
        for cp in g1:
            cp.wait()
        s1 = fire_scatters(1, ssem1)
        for cp in s0:
            cp.wait()
        for cp in s1:
            cp.wait()
        return 0

    lax.fori_loop(0, NB // 2, pair, 0)
    plsc.subcore_barrier()

    for i in range(NODES_PER_TILE // OUT_CHUNK):
        sl = pl.ds(nodeoff + i * OUT_CHUNK, OUT_CHUNK)
        pltpu.sync_copy(acc_sh.at[sl], stage)
        pltpu.sync_copy(stage, agg_out.at[c, sl])


def _tc1_body(x_ref, we_ref, be_ref, w1_ref, deg_ref, dinv_ref, p1_ref):
    deg = jnp.sum(deg_ref[...], axis=0) + 1.0
    di = lax.rsqrt(deg)
    h0 = jnp.dot(x_ref[...], we_ref[...], preferred_element_type=jnp.float32)
    h0 = jnp.maximum(h0 + be_ref[...][None, :], 0.0)
    p1 = di[:, None] * jnp.dot(h0, w1_ref[...], preferred_element_type=jnp.float32)
    dinv_ref[...] = di
    p1_ref[...] = p1


_tc1 = pl.pallas_call(
    _tc1_body,
    grid=(GRID,),
    in_specs=[
        pl.BlockSpec((BR, D), lambda i: (i, 0)),
        pl.BlockSpec((D, D), lambda i: (0, 0)),
        pl.BlockSpec((D,), lambda i: (0,)),
        pl.BlockSpec((D, D), lambda i: (0, 0)),
        pl.BlockSpec((NW, BR), lambda i: (0, i)),
    ],
    out_specs=[
        pl.BlockSpec((BR,), lambda i: (i,)),
        pl.BlockSpec((BR, D), lambda i: (i, 0)),
    ],
    out_shape=[
        jax.ShapeDtypeStruct((NP,), jnp.float32),
        jax.ShapeDtypeStruct((NP, D), jnp.float32),
    ],
)


def _tc2_body(agg_ref, p1_ref, dinv_ref, b1_ref, w2_ref, p2_ref):
    a = agg_ref[0] + agg_ref[1] + p1_ref[...]
    di = dinv_ref[...]
    h1 = jnp.maximum(di[:, None] * a + b1_ref[...][None, :], 0.0)
    p2_ref[...] = di[:, None] * jnp.dot(h1, w2_ref[...],
                                        preferred_element_type=jnp.float32)


_tc2 = pl.pallas_call(
    _tc2_body,
    grid=(GRID,),
    in_specs=[
        pl.BlockSpec((NC, BR, D), lambda i: (0, i, 0)),
        pl.BlockSpec((BR, D), lambda i: (i, 0)),
        pl.BlockSpec((BR,), lambda i: (i,)),
        pl.BlockSpec((D,), lambda i: (0,)),
        pl.BlockSpec((D, D), lambda i: (0, 0)),
    ],
    out_specs=pl.BlockSpec((BR, D), lambda i: (i, 0)),
    out_shape=jax.ShapeDtypeStruct((NP, D), jnp.float32),
)


def _tc3_body(agg_ref, p2_ref, dinv_ref, b2_ref, out_ref):
    a = agg_ref[0] + agg_ref[1] + p2_ref[...]
    out_ref[...] = dinv_ref[...][:, None] * a + b2_ref[...][None, :]


_tc3 = pl.pallas_call(
    _tc3_body,
    grid=(GRID,),
    in_specs=[
        pl.BlockSpec((NC, BR, D), lambda i: (0, i, 0)),
        pl.BlockSpec((BR, D), lambda i: (i, 0)),
        pl.BlockSpec((BR,), lambda i: (i,)),
        pl.BlockSpec((D,), lambda i: (0,)),
    ],
    out_specs=pl.BlockSpec((BR, D), lambda i: (i, 0)),
    out_shape=jax.ShapeDtypeStruct((NP, D), jnp.float32),
)


def kernel(x, edge_index, W_embed, b_embed, W1, b1, W2, b2):
    src = edge_index[0].astype(jnp.int32)
    dst = edge_index[1].astype(jnp.int32)
    pad = jnp.full((EP - E,), NP - 1, jnp.int32)
    src_p = jnp.concatenate([src, pad]).reshape(ROWS, 128)
    dst_p = jnp.concatenate([dst, pad]).reshape(ROWS, 128)
    dst_flat = dst_p.reshape(EP)

    x_p = jnp.pad(x, ((0, NP - N), (0, 0)))
    zeros1 = jnp.zeros((NP,), jnp.float32)
    zrow = jnp.zeros((OUT_CHUNK, D), jnp.float32)

    deg = _deg_kernel(dst_flat, zeros1)
    dinv, p1 = _tc1(x_p, W_embed, b_embed, W1, deg)
    agg1 = _agg_kernel(p1, src_p, dst_p, zrow)
    p2 = _tc2(agg1, p1, dinv, b1, W2)
    agg2 = _agg_kernel(p2, src_p, dst_p, zrow)
    out = _tc3(agg2, p2, dinv, b2)
    return out[:N]


# speedup vs baseline: 1.2355x; 1.2355x over previous
import functools

import jax
import jax.numpy as jnp
from jax import lax
from jax.experimental import pallas as pl
from jax.experimental.pallas import tpu as pltpu
from jax.experimental.pallas import tpu_sc as plsc

N = 100000
E = 3200000
D = 16

NC = 2
NS = 16
NW = NC * NS

NP = 102400
NODES_PER_TILE = NP // NS
OUT_CHUNK = 800

RPT = 784
ROWS = NW * RPT
EP = ROWS * 128
GRP = 8
NB = RPT // GRP

IDX_CHUNK = 3136
DEG_CHUNKS = (RPT * 128) // IDX_CHUNK

BR = 2048
GRID = NP // BR

_mesh = plsc.VectorSubcoreMesh(core_axis_name="c", subcore_axis_name="s",
                               num_cores=NC, num_subcores=NS)
_sc_params = pltpu.CompilerParams(needs_layout_passes=False,
                                  use_tc_tiling_on_sc=False)


@functools.partial(
    pl.kernel, mesh=_mesh, compiler_params=_sc_params, name="sc_degree",
    out_type=jax.ShapeDtypeStruct((NW, NP), jnp.float32),
    scratch_types=[
        pltpu.VMEM((NP,), jnp.float32),
        pltpu.VMEM((2, IDX_CHUNK), jnp.int32),
        pltpu.SemaphoreType.DMA,
    ],
)
def _deg_kernel(dst_hbm, zeros_hbm, deg_out, hist, ibuf, isem):
    c = lax.axis_index("c")
    s = lax.axis_index("s")
    wid = c * NS + s

    pltpu.sync_copy(zeros_hbm, hist)
    base = wid * (RPT * 128)
    ones = jnp.full((16,), 1.0, jnp.float32)

    def count(buf):
        def vec_body(i, _):
            for u in range(4):
                idxv = buf[pl.ds((i * 4 + u) * 16, 16)]
                plsc.addupdate_scatter(hist, [idxv], ones)
            return 0

        lax.fori_loop(0, IDX_CHUNK // 64, vec_body, 0)

    def chunk(k):
        return dst_hbm.at[pl.ds(base + k * IDX_CHUNK, IDX_CHUNK)]

    pltpu.sync_copy(chunk(0), ibuf.at[0])
    def pair_body(t, _):
        cpb = pltpu.async_copy(chunk(2 * t + 1), ibuf.at[1], isem)
        count(ibuf.at[0])
        cpb.wait()
        nxt = jnp.minimum(2 * t + 2, DEG_CHUNKS - 1)
        cpa = pltpu.async_copy(chunk(nxt), ibuf.at[0], isem)
        count(ibuf.at[1])
        cpa.wait()
        return 0

    lax.fori_loop(0, DEG_CHUNKS // 2, pair_body, 0)
    pltpu.sync_copy(hist, deg_out.at[wid])


@functools.partial(
    pl.kernel, mesh=_mesh, compiler_params=_sc_params, name="sc_gcn_agg",
    out_type=jax.ShapeDtypeStruct((NC, NP, D), jnp.float32),
    scratch_types=[
        pltpu.VMEM((GRP, 128), jnp.int32),
        pltpu.VMEM((GRP, 128), jnp.int32),
        pltpu.VMEM((GRP * 128, D), jnp.float32),
        pltpu.VMEM_SHARED((NP, D), jnp.float32),
        pltpu.SemaphoreType.DMA,
        pltpu.SemaphoreType.DMA,
    ],
)
def _agg_kernel(p_hbm, src_hbm, dst_hbm, zrow_hbm, agg_out,
                sbuf, dbuf, gbuf, acc_sh, gsem, ssem):
    c = lax.axis_index("c")
    s = lax.axis_index("s")
    wid = c * NS + s
    nodeoff = s * NODES_PER_TILE
    stage = gbuf.at[pl.ds(0, OUT_CHUNK)]

    pltpu.sync_copy(zrow_hbm, stage)
    for i in range(NODES_PER_TILE // OUT_CHUNK):
        pltpu.sync_copy(stage, acc_sh.at[pl.ds(nodeoff + i * OUT_CHUNK, OUT_CHUNK)])
    plsc.subcore_barrier()

    base_row = wid * RPT

    def batch(g, _):
        pltpu.sync_copy(src_hbm.at[pl.ds(base_row + g * GRP, GRP)], sbuf)
        pltpu.sync_copy(dst_hbm.at[pl.ds(base_row + g * GRP, GRP)], dbuf)
        cps = [pltpu.async_copy(p_hbm.at[sbuf.at[j]],
                                gbuf.at[pl.ds(j * 128, 128)], gsem)
               for j in range(GRP)]
        for cp in cps:
            cp.wait()
        cps = [pltpu.async_copy(gbuf.at[pl.ds(j * 128, 128)],
                                acc_sh.at[dbuf.at[j]], ssem, add=True)
               for j in range(GRP)]
        for cp in cps:
            cp.wait()
        return 0

    lax.fori_loop(0, NB, batch, 0)
    plsc.subcore_barrier()

    for i in range(NODES_PER_TILE // OUT_CHUNK):
        sl = pl.ds(nodeoff + i * OUT_CHUNK, OUT_CHUNK)
        pltpu.sync_copy(acc_sh.at[sl], stage)
        pltpu.sync_copy(stage, agg_out.at[c, sl])


def _tc1_body(x_ref, we_ref, be_ref, w1_ref, deg_ref, dinv_ref, p1_ref):
    deg = jnp.sum(deg_ref[...], axis=0) + 1.0
    di = lax.rsqrt(deg)
    h0 = jnp.dot(x_ref[...], we_ref[...], preferred_element_type=jnp.float32)
    h0 = jnp.maximum(h0 + be_ref[...][None, :], 0.0)
    p1 = di[:, None] * jnp.dot(h0, w1_ref[...], preferred_element_type=jnp.float32)
    dinv_ref[...] = di
    p1_ref[...] = p1


_tc1 = pl.pallas_call(
    _tc1_body,
    grid=(GRID,),
    in_specs=[
        pl.BlockSpec((BR, D), lambda i: (i, 0)),
        pl.BlockSpec((D, D), lambda i: (0, 0)),
        pl.BlockSpec((D,), lambda i: (0,)),
        pl.BlockSpec((D, D), lambda i: (0, 0)),
        pl.BlockSpec((NW, BR), lambda i: (0, i)),
    ],
    out_specs=[
        pl.BlockSpec((BR,), lambda i: (i,)),
        pl.BlockSpec((BR, D), lambda i: (i, 0)),
    ],
    out_shape=[
        jax.ShapeDtypeStruct((NP,), jnp.float32),
        jax.ShapeDtypeStruct((NP, D), jnp.float32),
    ],
)


def _tc2_body(agg_ref, p1_ref, dinv_ref, b1_ref, w2_ref, p2_ref):
    a = agg_ref[0] + agg_ref[1] + p1_ref[...]
    di = dinv_ref[...]
    h1 = jnp.maximum(di[:, None] * a + b1_ref[...][None, :], 0.0)
    p2_ref[...] = di[:, None] * jnp.dot(h1, w2_ref[...],
                                        preferred_element_type=jnp.float32)


_tc2 = pl.pallas_call(
    _tc2_body,
    grid=(GRID,),
    in_specs=[
        pl.BlockSpec((NC, BR, D), lambda i: (0, i, 0)),
        pl.BlockSpec((BR, D), lambda i: (i, 0)),
        pl.BlockSpec((BR,), lambda i: (i,)),
        pl.BlockSpec((D,), lambda i: (0,)),
        pl.BlockSpec((D, D), lambda i: (0, 0)),
    ],
    out_specs=pl.BlockSpec((BR, D), lambda i: (i, 0)),
    out_shape=jax.ShapeDtypeStruct((NP, D), jnp.float32),
)


def _tc3_body(agg_ref, p2_ref, dinv_ref, b2_ref, out_ref):
    a = agg_ref[0] + agg_ref[1] + p2_ref[...]
    out_ref[...] = dinv_ref[...][:, None] * a + b2_ref[...][None, :]


_tc3 = pl.pallas_call(
    _tc3_body,
    grid=(GRID,),
    in_specs=[
        pl.BlockSpec((NC, BR, D), lambda i: (0, i, 0)),
        pl.BlockSpec((BR, D), lambda i: (i, 0)),
        pl.BlockSpec((BR,), lambda i: (i,)),
        pl.BlockSpec((D,), lambda i: (0,)),
    ],
    out_specs=pl.BlockSpec((BR, D), lambda i: (i, 0)),
    out_shape=jax.ShapeDtypeStruct((NP, D), jnp.float32),
)


def kernel(x, edge_index, W_embed, b_embed, W1, b1, W2, b2):
    src = edge_index[0].astype(jnp.int32)
    dst = edge_index[1].astype(jnp.int32)
    pad = jnp.full((EP - E,), NP - 1, jnp.int32)
    src_p = jnp.concatenate([src, pad]).reshape(ROWS, 128)
    dst_p = jnp.concatenate([dst, pad]).reshape(ROWS, 128)
    dst_flat = dst_p.reshape(EP)

    x_p = jnp.pad(x, ((0, NP - N), (0, 0)))
    zeros1 = jnp.zeros((NP,), jnp.float32)
    zrow = jnp.zeros((OUT_CHUNK, D), jnp.float32)

    deg = _deg_kernel(dst_flat, zeros1)
    dinv, p1 = _tc1(x_p, W_embed, b_embed, W1, deg)
    agg1 = _agg_kernel(p1, src_p, dst_p, zrow)
    p2 = _tc2(agg1, p1, dinv, b1, W2)
    agg2 = _agg_kernel(p2, src_p, dst_p, zrow)
    out = _tc3(agg2, p2, dinv, b2)
    return out[:N]
